# baseline (speedup 1.0000x reference)
import functools

import jax
import jax.numpy as jnp
from jax import lax
from jax.experimental import pallas as pl
from jax.experimental.pallas import tpu as pltpu

_ZS = {'traffic': [1385.4058837890625, 859.8118896484375], 'packets': [1.4015231132507324, 0.8932565450668335], 'eq_lambda': [1350.97119140625, 858.316162109375], 'avg_pkts_lambda': [0.9117304086685181, 0.9723503589630127], 'exp_max_factor': [6.663637638092041, 4.715115070343018], 'pkts_lambda_on': [0.9116322994232178, 1.651275396347046], 'avg_t_off': [1.6649284362792969, 2.356407403945923], 'avg_t_on': [1.6649284362792969, 2.356407403945923], 'ar_a': [0.0, 1.0], 'sigma': [0.0, 1.0], 'capacity': [27611.091796875, 20090.62109375], 'queue_size': [30259.10546875, 21410.095703125]}

_BLK = 1024


def _pad_rows(x, n):
    if x.shape[0] == n:
        return x
    return jnp.pad(x, ((0, n - x.shape[0]),) + ((0, 0),) * (x.ndim - 1))


def _rup(n, m):
    return -(-n // m) * m


def _embed_body(x_ref, w1_ref, b1_ref, w2_ref, b2_ref, o_ref):
    h = jnp.maximum(jnp.dot(x_ref[...], w1_ref[...],
                            preferred_element_type=jnp.float32) + b1_ref[...], 0.0)
    o_ref[...] = jnp.maximum(jnp.dot(h, w2_ref[...],
                                     preferred_element_type=jnp.float32) + b2_ref[...], 0.0)


def _embed(x, w1, b1, w2, b2):
    n, fin = x.shape
    sd = w1.shape[1]
    grid = n // _BLK
    return pl.pallas_call(
        _embed_body,
        grid=(grid,),
        in_specs=[
            pl.BlockSpec((_BLK, fin), lambda i: (i, 0)),
            pl.BlockSpec((w1.shape[0], sd), lambda i: (0, 0)),
            pl.BlockSpec((1, sd), lambda i: (0, 0)),
            pl.BlockSpec((sd, sd), lambda i: (0, 0)),
            pl.BlockSpec((1, sd), lambda i: (0, 0)),
        ],
        out_specs=pl.BlockSpec((_BLK, sd), lambda i: (i, 0)),
        out_shape=jax.ShapeDtypeStruct((n, sd), jnp.float32),
    )(x, w1, b1.reshape(1, -1), w2, b2.reshape(1, -1))


def _lstm_math(x, h, c, w, b):
    z = jnp.dot(jnp.concatenate([x, h], axis=1), w,
                preferred_element_type=jnp.float32) + b
    sd = h.shape[1]
    i = z[:, :sd]
    f = z[:, sd:2 * sd]
    g = z[:, 2 * sd:3 * sd]
    o = z[:, 3 * sd:]
    c2 = jax.nn.sigmoid(f) * c + jax.nn.sigmoid(i) * jnp.tanh(g)
    h2 = jax.nn.sigmoid(o) * jnp.tanh(c2)
    return h2, c2


def _path_lstm_body(xq_ref, xl_ref, h0_ref, c0_ref, w_ref, b_ref, seq_ref, c_ref):
    t_steps = xq_ref.shape[0]
    h = h0_ref[...]
    c = c0_ref[...]
    w = w_ref[...]
    b = b_ref[...]
    seq_ref[0] = h
    for t in range(t_steps):
        x = jnp.concatenate([xq_ref[t], xl_ref[t]], axis=1)
        h, c = _lstm_math(x, h, c, w, b)
        seq_ref[t + 1] = h
    c_ref[...] = c


def _path_lstm(xq, xl, h0, c0, w, b):
    t_steps, n, sd = xq.shape
    grid = n // _BLK
    return pl.pallas_call(
        _path_lstm_body,
        grid=(grid,),
        in_specs=[
            pl.BlockSpec((t_steps, _BLK, sd), lambda i: (0, i, 0)),
            pl.BlockSpec((t_steps, _BLK, sd), lambda i: (0, i, 0)),
            pl.BlockSpec((_BLK, sd), lambda i: (i, 0)),
            pl.BlockSpec((_BLK, sd), lambda i: (i, 0)),
            pl.BlockSpec(w.shape, lambda i: (0, 0)),
            pl.BlockSpec((1, 4 * sd), lambda i: (0, 0)),
        ],
        out_specs=[
            pl.BlockSpec((t_steps + 1, _BLK, sd), lambda i: (0, i, 0)),
            pl.BlockSpec((_BLK, sd), lambda i: (i, 0)),
        ],
        out_shape=[
            jax.ShapeDtypeStruct((t_steps + 1, n, sd), jnp.float32),
            jax.ShapeDtypeStruct((n, sd), jnp.float32),
        ],
    )(xq, xl, h0, c0, w, b.reshape(1, -1))


def _step_lstm_body(x_ref, h0_ref, c0_ref, w_ref, b_ref, h_ref, c_ref):
    h, c = _lstm_math(x_ref[...], h0_ref[...], c0_ref[...], w_ref[...], b_ref[...])
    h_ref[...] = h
    c_ref[...] = c


def _step_lstm(x, h0, c0, w, b):
    n, sd = h0.shape
    grid = n // _BLK
    return pl.pallas_call(
        _step_lstm_body,
        grid=(grid,),
        in_specs=[
            pl.BlockSpec((_BLK, sd), lambda i: (i, 0)),
            pl.BlockSpec((_BLK, sd), lambda i: (i, 0)),
            pl.BlockSpec((_BLK, sd), lambda i: (i, 0)),
            pl.BlockSpec(w.shape, lambda i: (0, 0)),
            pl.BlockSpec((1, 4 * sd), lambda i: (0, 0)),
        ],
        out_specs=[
            pl.BlockSpec((_BLK, sd), lambda i: (i, 0)),
            pl.BlockSpec((_BLK, sd), lambda i: (i, 0)),
        ],
        out_shape=[
            jax.ShapeDtypeStruct((n, sd), jnp.float32),
            jax.ShapeDtypeStruct((n, sd), jnp.float32),
        ],
    )(x, h0, c0, w, b.reshape(1, -1))


def _seq_lstm_body(x_ref, h0_ref, c0_ref, w_ref, b_ref, h_ref, c_ref):
    t_steps = x_ref.shape[0]
    h = h0_ref[...]
    c = c0_ref[...]
    w = w_ref[...]
    b = b_ref[...]
    for t in range(t_steps):
        h, c = _lstm_math(x_ref[t], h, c, w, b)
    h_ref[...] = h
    c_ref[...] = c


def _seq_lstm(x, h0, c0, w, b):
    t_steps, n, sd = x.shape
    grid = n // _BLK
    return pl.pallas_call(
        _seq_lstm_body,
        grid=(grid,),
        in_specs=[
            pl.BlockSpec((t_steps, _BLK, sd), lambda i: (0, i, 0)),
            pl.BlockSpec((_BLK, sd), lambda i: (i, 0)),
            pl.BlockSpec((_BLK, sd), lambda i: (i, 0)),
            pl.BlockSpec(w.shape, lambda i: (0, 0)),
            pl.BlockSpec((1, 4 * sd), lambda i: (0, 0)),
        ],
        out_specs=[
            pl.BlockSpec((_BLK, sd), lambda i: (i, 0)),
            pl.BlockSpec((_BLK, sd), lambda i: (i, 0)),
        ],
        out_shape=[
            jax.ShapeDtypeStruct((n, sd), jnp.float32),
            jax.ShapeDtypeStruct((n, sd), jnp.float32),
        ],
    )(x, h0, c0, w, b.reshape(1, -1))


def _readout_body(h_ref, w1_ref, b1_ref, w2_ref, b2_ref, w3_ref, b3_ref, o_ref):
    h = jnp.maximum(jnp.dot(h_ref[...], w1_ref[...],
                            preferred_element_type=jnp.float32) + b1_ref[...], 0.0)
    h = jnp.maximum(jnp.dot(h, w2_ref[...],
                            preferred_element_type=jnp.float32) + b2_ref[...], 0.0)
    o_ref[...] = jax.nn.sigmoid(
        jnp.dot(h, w3_ref[...], preferred_element_type=jnp.float32) + b3_ref[...])


def _readout(h, w1, b1, w2, b2, w3, b3):
    n, sd = h.shape
    hd = w1.shape[1]
    grid = n // _BLK
    return pl.pallas_call(
        _readout_body,
        grid=(grid,),
        in_specs=[
            pl.BlockSpec((_BLK, sd), lambda i: (i, 0)),
            pl.BlockSpec((sd, hd), lambda i: (0, 0)),
            pl.BlockSpec((1, hd), lambda i: (0, 0)),
            pl.BlockSpec((hd, hd), lambda i: (0, 0)),
            pl.BlockSpec((1, hd), lambda i: (0, 0)),
            pl.BlockSpec((hd, 1), lambda i: (0, 0)),
            pl.BlockSpec((1, 1), lambda i: (0, 0)),
        ],
        out_specs=pl.BlockSpec((_BLK, 1), lambda i: (i, 0)),
        out_shape=jax.ShapeDtypeStruct((n, 1), jnp.float32),
    )(h, w1, b1.reshape(1, -1), w2, b2.reshape(1, -1), w3, b3.reshape(1, -1))


def kernel(traffic, packets, length, model, eq_lambda, avg_pkts_lambda,
           exp_max_factor, pkts_lambda_on, avg_t_off, avg_t_on, ar_a, sigma,
           capacity, policy, queue_size, priority, weight, queue_to_path,
           link_to_path, path_to_link, path_to_queue, queue_to_link,
           Wpe1, bpe1, Wpe2, bpe2, Wle1, ble1, Wle2, ble2,
           Wqe1, bqe1, Wqe2, bqe2, Wp, Up, bp, Wq, Uq, bq, Wl, Ul, bl,
           Wr1, br1, Wr2, br2, Wr3, br3):
    f32 = jnp.float32
    np0 = traffic.shape[0]
    nl0 = capacity.shape[0]
    nq0 = queue_size.shape[0]
    NP = _rup(np0, _BLK)
    NL = _rup(nl0, _BLK)
    NQ = _rup(nq0, _BLK)
    T = queue_to_path.shape[1]

    def zn(x, name):
        return (x - _ZS[name][0]) / _ZS[name][1]

    model_oh = jax.nn.one_hot(model, 7, dtype=f32)
    policy_oh = jax.nn.one_hot(policy, 4, dtype=f32)
    priority_oh = jax.nn.one_hot(priority, 3, dtype=f32)

    path_in = jnp.concatenate([
        zn(traffic, 'traffic'), zn(packets, 'packets'), model_oh,
        zn(eq_lambda, 'eq_lambda'), zn(avg_pkts_lambda, 'avg_pkts_lambda'),
        zn(exp_max_factor, 'exp_max_factor'), zn(pkts_lambda_on, 'pkts_lambda_on'),
        zn(avg_t_off, 'avg_t_off'), zn(avg_t_on, 'avg_t_on'),
        zn(ar_a, 'ar_a'), zn(sigma, 'sigma')], axis=1)
    path_in = _pad_rows(path_in, NP)

    pgt = jnp.take(traffic, path_to_link[:, :, 0], axis=0)
    load = jnp.sum(pgt, axis=1) / capacity
    link_in = _pad_rows(jnp.concatenate([load, policy_oh], axis=1), NL)

    queue_in = _pad_rows(jnp.concatenate(
        [zn(queue_size, 'queue_size'), priority_oh, weight], axis=1), NQ)

    path_h = _embed(path_in, Wpe1, bpe1, Wpe2, bpe2)
    link_h = _embed(link_in, Wle1, ble1, Wle2, ble2)
    queue_h = _embed(queue_in, Wqe1, bqe1, Wqe2, bqe2)
    path_c = jnp.zeros_like(path_h)
    link_c = jnp.zeros_like(link_h)
    queue_c = jnp.zeros_like(queue_h)

    Wp_all = jnp.concatenate([Wp, Up], axis=0)
    Wq_all = jnp.concatenate([Wq, Uq], axis=0)
    Wl_all = jnp.concatenate([Wl, Ul], axis=0)

    q2pT = _pad_rows(queue_to_path, NP).T
    l2pT = _pad_rows(link_to_path, NP).T
    p2q_flat = _pad_rows(path_to_queue[..., 1] * NP + path_to_queue[..., 0], NQ)
    q2lT = _pad_rows(queue_to_link, NL).T

    n_pq = p2q_flat.shape[1]

    for _ in range(8):
        xq = jnp.take(queue_h, q2pT, axis=0)
        xl = jnp.take(link_h, l2pT, axis=0)
        path_seq, path_c = _path_lstm(xq, xl, path_h, path_c, Wp_all, bp)
        path_h = path_seq[T]
        pg = jnp.take(path_seq.reshape(-1, path_seq.shape[-1]),
                      p2q_flat.reshape(-1), axis=0)
        path_sum = jnp.sum(pg.reshape(NQ, n_pq, -1), axis=1)
        queue_h, queue_c = _step_lstm(path_sum, queue_h, queue_c, Wq_all, bq)
        xql = jnp.take(queue_h, q2lT, axis=0)
        link_h, link_c = _seq_lstm(xql, link_h, link_c, Wl_all, bl)

    out = _readout(path_h, Wr1, br1, Wr2, br2, Wr3, br3)
    return out[:np0]


# speedup vs baseline: 3.5109x; 3.5109x over previous
import functools

import jax
import jax.numpy as jnp
from jax import lax
from jax.experimental import pallas as pl
from jax.experimental.pallas import tpu as pltpu
from jax.experimental.pallas import tpu_sc as plsc

_NC, _NS = 2, 16
_NW = _NC * _NS

_ZS = {'traffic': [1385.4058837890625, 859.8118896484375], 'packets': [1.4015231132507324, 0.8932565450668335], 'eq_lambda': [1350.97119140625, 858.316162109375], 'avg_pkts_lambda': [0.9117304086685181, 0.9723503589630127], 'exp_max_factor': [6.663637638092041, 4.715115070343018], 'pkts_lambda_on': [0.9116322994232178, 1.651275396347046], 'avg_t_off': [1.6649284362792969, 2.356407403945923], 'avg_t_on': [1.6649284362792969, 2.356407403945923], 'ar_a': [0.0, 1.0], 'sigma': [0.0, 1.0], 'capacity': [27611.091796875, 20090.62109375], 'queue_size': [30259.10546875, 21410.095703125]}

_BLK = 1024


def _pad_rows(x, n):
    if x.shape[0] == n:
        return x
    return jnp.pad(x, ((0, n - x.shape[0]),) + ((0, 0),) * (x.ndim - 1))


def _rup(n, m):
    return -(-n // m) * m


def _sc_mesh():
    return plsc.VectorSubcoreMesh(core_axis_name="c", subcore_axis_name="s")


_NBUF = 4


def _pick_chunk(m_w, row_quant=8):
    for ch in range(128, 0, -row_quant):
        if m_w % ch == 0 and (m_w // ch) % _NBUF == 0:
            return ch
    raise ValueError(f"no chunk for {m_w}")


def _sc_gather_rows(table, idx_flat):
    v, d = table.shape
    m = idx_flat.shape[0]
    assert m % (_NW * 8) == 0
    m_w = m // _NW
    ch = _pick_chunk(m_w)
    kch = m_w // ch
    idx3 = idx_flat.reshape(_NW, kch, ch)

    @functools.partial(
        pl.kernel, mesh=_sc_mesh(),
        compiler_params=pltpu.CompilerParams(needs_layout_passes=False, use_tc_tiling_on_sc=False),
        out_type=jax.ShapeDtypeStruct((m, d), jnp.float32),
        scratch_types=(
            [pltpu.VMEM((kch, ch), jnp.int32)]
            + [pltpu.VMEM((ch, d), jnp.float32)] * _NBUF
            + [pltpu.SemaphoreType.DMA] * _NBUF
        ),
    )
    def k(table_h, idx_h, out_h, idx_v, *bs):
        bufs, sems = bs[:_NBUF], bs[_NBUF:]
        wid = lax.axis_index("s") * _NC + lax.axis_index("c")
        base = wid * m_w
        pltpu.sync_copy(idx_h.at[wid], idx_v)
        for b in range(_NBUF):
            pltpu.async_copy(table_h.at[idx_v.at[b]], bufs[b], sems[b])

        def step(i, carry):
            k0 = _NBUF * i
            for b in range(_NBUF):
                kk = k0 + b
                pltpu.make_async_copy(
                    table_h.at[idx_v.at[kk]], bufs[b], sems[b]).wait()
                pltpu.sync_copy(bufs[b], out_h.at[pl.ds(base + kk * ch, ch)])

                @pl.when(kk + _NBUF < kch)
                def _():
                    pltpu.async_copy(
                        table_h.at[idx_v.at[kk + _NBUF]], bufs[b], sems[b])

            return carry

        lax.fori_loop(0, kch // _NBUF, step, 0)

    return k(table, idx3)


def _sc_gather_sum(table, idx_flat, n_out, n_per):
    v, d = table.shape
    m = idx_flat.shape[0]
    assert m == n_out * n_per and m % _NW == 0
    m_w = m // _NW
    q_w = n_out // _NW
    ch = _pick_chunk(m_w, row_quant=n_per)
    qpc = ch // n_per
    kch = m_w // ch
    idx3 = idx_flat.reshape(_NW, kch, ch)
    half = d // 2

    @functools.partial(
        pl.kernel, mesh=_sc_mesh(),
        compiler_params=pltpu.CompilerParams(needs_layout_passes=False, use_tc_tiling_on_sc=False),
        out_type=jax.ShapeDtypeStruct((n_out, d), jnp.float32),
        scratch_types=(
            [pltpu.VMEM((kch, ch), jnp.int32)]
            + [pltpu.VMEM((ch, d), jnp.float32)] * _NBUF
            + [pltpu.VMEM((q_w, d), jnp.float32)]
            + [pltpu.SemaphoreType.DMA] * _NBUF
        ),
    )
    def k(table_h, idx_h, out_h, idx_v, *bs):
        bufs, out_v, sems = bs[:_NBUF], bs[_NBUF], bs[_NBUF + 1:]
        wid = lax.axis_index("s") * _NC + lax.axis_index("c")
        pltpu.sync_copy(idx_h.at[wid], idx_v)
        for b in range(_NBUF):
            pltpu.async_copy(table_h.at[idx_v.at[b]], bufs[b], sems[b])

        def reduce_chunk(buf, kk):
            def qstep(q, carry):
                acc0 = jnp.zeros((16,), jnp.float32)
                acc1 = jnp.zeros((16,), jnp.float32)
                for j in range(n_per):
                    acc0 = acc0 + buf[q * n_per + j, 0:16]
                    acc1 = acc1 + buf[q * n_per + j, 16:32]
                qq = kk * qpc + q
                out_v[qq, 0:16] = acc0
                out_v[qq, 16:32] = acc1
                return carry
            lax.fori_loop(0, qpc, qstep, 0)

        def step(i, carry):
            k0 = _NBUF * i
            for b in range(_NBUF):
                kk = k0 + b
                pltpu.make_async_copy(
                    table_h.at[idx_v.at[kk]], bufs[b], sems[b]).wait()
                reduce_chunk(bufs[b], kk)

                @pl.when(kk + _NBUF < kch)
                def _():
                    pltpu.async_copy(
                        table_h.at[idx_v.at[kk + _NBUF]], bufs[b], sems[b])

            return carry

        lax.fori_loop(0, kch // _NBUF, step, 0)
        pltpu.sync_copy(out_v, out_h.at[pl.ds(wid * q_w, q_w)])

    assert half == 16
    return k(table, idx3)


def _sc_link_load(traffic_flat, p2l0_jmaj, capacity_flat, n_links, n_per):
    npth = traffic_flat.shape[0]
    l_w = n_links // _NW
    idx2 = p2l0_jmaj.reshape(_NW, n_per * l_w)
    cap2 = capacity_flat.reshape(_NW, l_w)

    @functools.partial(
        pl.kernel, mesh=_sc_mesh(),
        compiler_params=pltpu.CompilerParams(needs_layout_passes=False, use_tc_tiling_on_sc=False),
        out_type=jax.ShapeDtypeStruct((n_links,), jnp.float32),
        scratch_types=[
            pltpu.VMEM((npth,), jnp.float32),
            pltpu.VMEM((n_per * l_w,), jnp.int32),
            pltpu.VMEM((l_w,), jnp.float32),
            pltpu.VMEM((l_w,), jnp.float32),
        ],
    )
    def k(tr_h, idx_h, cap_h, out_h, tr_v, idx_v, cap_v, out_v):
        wid = lax.axis_index("s") * _NC + lax.axis_index("c")
        pltpu.sync_copy(tr_h, tr_v)
        pltpu.sync_copy(idx_h.at[wid], idx_v)
        pltpu.sync_copy(cap_h.at[wid], cap_v)

        def grp(g, carry):
            acc = jnp.zeros((16,), jnp.float32)
            for j in range(n_per):
                ii = idx_v[pl.ds(j * l_w + g * 16, 16)]
                acc = acc + plsc.load_gather(tr_v, [ii])
            out_v[pl.ds(g * 16, 16)] = acc / cap_v[pl.ds(g * 16, 16)]
            return carry

        lax.fori_loop(0, l_w // 16, grp, 0)
        pltpu.sync_copy(out_v, out_h.at[pl.ds(wid * l_w, l_w)])

    return k(traffic_flat, idx2, cap2)


def _embed_body(x_ref, w1_ref, b1_ref, w2_ref, b2_ref, o_ref):
    h = jnp.maximum(jnp.dot(x_ref[...], w1_ref[...],
                            preferred_element_type=jnp.float32) + b1_ref[...], 0.0)
    o_ref[...] = jnp.maximum(jnp.dot(h, w2_ref[...],
                                     preferred_element_type=jnp.float32) + b2_ref[...], 0.0)


def _embed(x, w1, b1, w2, b2):
    n, fin = x.shape
    sd = w1.shape[1]
    grid = n // _BLK
    return pl.pallas_call(
        _embed_body,
        grid=(grid,),
        in_specs=[
            pl.BlockSpec((_BLK, fin), lambda i: (i, 0)),
            pl.BlockSpec((w1.shape[0], sd), lambda i: (0, 0)),
            pl.BlockSpec((1, sd), lambda i: (0, 0)),
            pl.BlockSpec((sd, sd), lambda i: (0, 0)),
            pl.BlockSpec((1, sd), lambda i: (0, 0)),
        ],
        out_specs=pl.BlockSpec((_BLK, sd), lambda i: (i, 0)),
        out_shape=jax.ShapeDtypeStruct((n, sd), jnp.float32),
    )(x, w1, b1.reshape(1, -1), w2, b2.reshape(1, -1))


def _lstm_math(x, h, c, w, b):
    z = jnp.dot(jnp.concatenate([x, h], axis=1), w,
                preferred_element_type=jnp.float32) + b
    sd = h.shape[1]
    i = z[:, :sd]
    f = z[:, sd:2 * sd]
    g = z[:, 2 * sd:3 * sd]
    o = z[:, 3 * sd:]
    c2 = jax.nn.sigmoid(f) * c + jax.nn.sigmoid(i) * jnp.tanh(g)
    h2 = jax.nn.sigmoid(o) * jnp.tanh(c2)
    return h2, c2


def _path_lstm_body(x_ref, h0_ref, c0_ref, w_ref, b_ref, seq_ref, c_ref):
    t_steps = x_ref.shape[0] // 2
    h = h0_ref[...]
    c = c0_ref[...]
    w = w_ref[...]
    b = b_ref[...]
    seq_ref[0] = h
    for t in range(t_steps):
        x = jnp.concatenate([x_ref[t], x_ref[t_steps + t]], axis=1)
        h, c = _lstm_math(x, h, c, w, b)
        seq_ref[t + 1] = h
    c_ref[...] = c


def _path_lstm(x, h0, c0, w, b):
    t2, n, sd = x.shape
    t_steps = t2 // 2
    grid = n // _BLK
    return pl.pallas_call(
        _path_lstm_body,
        grid=(grid,),
        in_specs=[
            pl.BlockSpec((t2, _BLK, sd), lambda i: (0, i, 0)),
            pl.BlockSpec((_BLK, sd), lambda i: (i, 0)),
            pl.BlockSpec((_BLK, sd), lambda i: (i, 0)),
            pl.BlockSpec(w.shape, lambda i: (0, 0)),
            pl.BlockSpec((1, 4 * sd), lambda i: (0, 0)),
        ],
        out_specs=[
            pl.BlockSpec((t_steps + 1, _BLK, sd), lambda i: (0, i, 0)),
            pl.BlockSpec((_BLK, sd), lambda i: (i, 0)),
        ],
        out_shape=[
            jax.ShapeDtypeStruct((t_steps + 1, n, sd), jnp.float32),
            jax.ShapeDtypeStruct((n, sd), jnp.float32),
        ],
    )(x, h0, c0, w, b.reshape(1, -1))


def _step_lstm_body(x_ref, h0_ref, c0_ref, w_ref, b_ref, h_ref, c_ref):
    h, c = _lstm_math(x_ref[...], h0_ref[...], c0_ref[...], w_ref[...], b_ref[...])
    h_ref[...] = h
    c_ref[...] = c


def _step_lstm(x, h0, c0, w, b):
    n, sd = h0.shape
    grid = n // _BLK
    return pl.pallas_call(
        _step_lstm_body,
        grid=(grid,),
        in_specs=[
            pl.BlockSpec((_BLK, sd), lambda i: (i, 0)),
            pl.BlockSpec((_BLK, sd), lambda i: (i, 0)),
            pl.BlockSpec((_BLK, sd), lambda i: (i, 0)),
            pl.BlockSpec(w.shape, lambda i: (0, 0)),
            pl.BlockSpec((1, 4 * sd), lambda i: (0, 0)),
        ],
        out_specs=[
            pl.BlockSpec((_BLK, sd), lambda i: (i, 0)),
            pl.BlockSpec((_BLK, sd), lambda i: (i, 0)),
        ],
        out_shape=[
            jax.ShapeDtypeStruct((n, sd), jnp.float32),
            jax.ShapeDtypeStruct((n, sd), jnp.float32),
        ],
    )(x, h0, c0, w, b.reshape(1, -1))


def _seq_lstm_body(x_ref, h0_ref, c0_ref, w_ref, b_ref, h_ref, c_ref):
    t_steps = x_ref.shape[0]
    h = h0_ref[...]
    c = c0_ref[...]
    w = w_ref[...]
    b = b_ref[...]
    for t in range(t_steps):
        h, c = _lstm_math(x_ref[t], h, c, w, b)
    h_ref[...] = h
    c_ref[...] = c


def _seq_lstm(x, h0, c0, w, b):
    t_steps, n, sd = x.shape
    grid = n // _BLK
    return pl.pallas_call(
        _seq_lstm_body,
        grid=(grid,),
        in_specs=[
            pl.BlockSpec((t_steps, _BLK, sd), lambda i: (0, i, 0)),
            pl.BlockSpec((_BLK, sd), lambda i: (i, 0)),
            pl.BlockSpec((_BLK, sd), lambda i: (i, 0)),
            pl.BlockSpec(w.shape, lambda i: (0, 0)),
            pl.BlockSpec((1, 4 * sd), lambda i: (0, 0)),
        ],
        out_specs=[
            pl.BlockSpec((_BLK, sd), lambda i: (i, 0)),
            pl.BlockSpec((_BLK, sd), lambda i: (i, 0)),
        ],
        out_shape=[
            jax.ShapeDtypeStruct((n, sd), jnp.float32),
            jax.ShapeDtypeStruct((n, sd), jnp.float32),
        ],
    )(x, h0, c0, w, b.reshape(1, -1))


def _readout_body(h_ref, w1_ref, b1_ref, w2_ref, b2_ref, w3_ref, b3_ref, o_ref):
    h = jnp.maximum(jnp.dot(h_ref[...], w1_ref[...],
                            preferred_element_type=jnp.float32) + b1_ref[...], 0.0)
    h = jnp.maximum(jnp.dot(h, w2_ref[...],
                            preferred_element_type=jnp.float32) + b2_ref[...], 0.0)
    o_ref[...] = jax.nn.sigmoid(
        jnp.dot(h, w3_ref[...], preferred_element_type=jnp.float32) + b3_ref[...])


def _readout(h, w1, b1, w2, b2, w3, b3):
    n, sd = h.shape
    hd = w1.shape[1]
    grid = n // _BLK
    return pl.pallas_call(
        _readout_body,
        grid=(grid,),
        in_specs=[
            pl.BlockSpec((_BLK, sd), lambda i: (i, 0)),
            pl.BlockSpec((sd, hd), lambda i: (0, 0)),
            pl.BlockSpec((1, hd), lambda i: (0, 0)),
            pl.BlockSpec((hd, hd), lambda i: (0, 0)),
            pl.BlockSpec((1, hd), lambda i: (0, 0)),
            pl.BlockSpec((hd, 1), lambda i: (0, 0)),
            pl.BlockSpec((1, 1), lambda i: (0, 0)),
        ],
        out_specs=pl.BlockSpec((_BLK, 1), lambda i: (i, 0)),
        out_shape=jax.ShapeDtypeStruct((n, 1), jnp.float32),
    )(h, w1, b1.reshape(1, -1), w2, b2.reshape(1, -1), w3, b3.reshape(1, -1))


def kernel(traffic, packets, length, model, eq_lambda, avg_pkts_lambda,
           exp_max_factor, pkts_lambda_on, avg_t_off, avg_t_on, ar_a, sigma,
           capacity, policy, queue_size, priority, weight, queue_to_path,
           link_to_path, path_to_link, path_to_queue, queue_to_link,
           Wpe1, bpe1, Wpe2, bpe2, Wle1, ble1, Wle2, ble2,
           Wqe1, bqe1, Wqe2, bqe2, Wp, Up, bp, Wq, Uq, bq, Wl, Ul, bl,
           Wr1, br1, Wr2, br2, Wr3, br3):
    f32 = jnp.float32
    np0 = traffic.shape[0]
    nl0 = capacity.shape[0]
    nq0 = queue_size.shape[0]
    NP = _rup(np0, _BLK)
    NL = _rup(nl0, _BLK)
    NQ = _rup(nq0, _BLK)
    T = queue_to_path.shape[1]

    def zn(x, name):
        return (x - _ZS[name][0]) / _ZS[name][1]

    model_oh = jax.nn.one_hot(model, 7, dtype=f32)
    policy_oh = jax.nn.one_hot(policy, 4, dtype=f32)
    priority_oh = jax.nn.one_hot(priority, 3, dtype=f32)

    path_in = jnp.concatenate([
        zn(traffic, 'traffic'), zn(packets, 'packets'), model_oh,
        zn(eq_lambda, 'eq_lambda'), zn(avg_pkts_lambda, 'avg_pkts_lambda'),
        zn(exp_max_factor, 'exp_max_factor'), zn(pkts_lambda_on, 'pkts_lambda_on'),
        zn(avg_t_off, 'avg_t_off'), zn(avg_t_on, 'avg_t_on'),
        zn(ar_a, 'ar_a'), zn(sigma, 'sigma')], axis=1)
    path_in = _pad_rows(path_in, NP)

    pl_per = path_to_link.shape[1]
    p2l0 = _pad_rows(path_to_link[:, :, 0], NL)
    p2l0 = p2l0.reshape(_NW, NL // _NW, pl_per).transpose(0, 2, 1)
    cap_pad = jnp.concatenate(
        [capacity[:, 0], jnp.ones((NL - nl0,), f32)])
    load = _sc_link_load(_pad_rows(traffic[:, 0], NP), p2l0, cap_pad, NL, pl_per)
    link_in = _pad_rows(
        jnp.concatenate([load[:nl0, None], policy_oh], axis=1), NL)

    queue_in = _pad_rows(jnp.concatenate(
        [zn(queue_size, 'queue_size'), priority_oh, weight], axis=1), NQ)

    path_h = _embed(path_in, Wpe1, bpe1, Wpe2, bpe2)
    link_h = _embed(link_in, Wle1, ble1, Wle2, ble2)
    queue_h = _embed(queue_in, Wqe1, bqe1, Wqe2, bqe2)
    path_c = jnp.zeros_like(path_h)
    link_c = jnp.zeros_like(link_h)
    queue_c = jnp.zeros_like(queue_h)

    Wp_all = jnp.concatenate([Wp, Up], axis=0)
    Wq_all = jnp.concatenate([Wq, Uq], axis=0)
    Wl_all = jnp.concatenate([Wl, Ul], axis=0)

    q2pT = _pad_rows(queue_to_path, NP).T
    l2pT = _pad_rows(link_to_path, NP).T
    x_idx = jnp.concatenate(
        [q2pT.reshape(-1), l2pT.reshape(-1) + NQ])
    p2q_flat = _pad_rows(path_to_queue[..., 1] * NP + path_to_queue[..., 0],
                         NQ).reshape(-1)
    q2l_flat = _pad_rows(queue_to_link, NL).T.reshape(-1)
    n_pq = path_to_queue.shape[1]
    QL = queue_to_link.shape[1]

    for _ in range(8):
        table = jnp.concatenate([queue_h, link_h], axis=0)
        x = _sc_gather_rows(table, x_idx).reshape(2 * T, NP, -1)
        path_seq, path_c = _path_lstm(x, path_h, path_c, Wp_all, bp)
        path_h = path_seq[T]
        path_sum = _sc_gather_sum(
            path_seq.reshape(-1, path_seq.shape[-1]), p2q_flat, NQ, n_pq)
        queue_h, queue_c = _step_lstm(path_sum, queue_h, queue_c, Wq_all, bq)
        xql = _sc_gather_rows(queue_h, q2l_flat).reshape(QL, NL, -1)
        link_h, link_c = _seq_lstm(xql, link_h, link_c, Wl_all, bl)

    out = _readout(path_h, Wr1, br1, Wr2, br2, Wr3, br3)
    return out[:np0]
